# baseline (speedup 1.0000x reference)
import functools

import jax
import jax.numpy as jnp
from jax import lax
from jax.experimental import pallas as pl
from jax.experimental.pallas import tpu as pltpu
from jax.experimental.pallas import tpu_sc as plsc

_H, _W = 181, 360
_N_GRID = _H * _W
_N_MESH = 10242
_C_IN = 128
_C_OUT = 128
_HID = 256
_L = 4

_NP_GRID = 65536
_NP_MESH = 10752
_NP_EG = 130560
_NP_EM = 40960

_RBLK = 512
_INTERPRET = False
_USE_SC = True


def _pad_rows(a, n):
    return jnp.pad(a, ((0, n - a.shape[0]), (0, 0)))


def _pad_cols(a, n):
    return jnp.pad(a, ((0, 0), (0, n - a.shape[1])))


def _pack_i32(t):
    n, d = t.shape
    return lax.bitcast_convert_type(t.reshape(n, d // 2, 2), jnp.int32)


def _unpack_bf16(t):
    n, d2 = t.shape
    return lax.bitcast_convert_type(t, jnp.bfloat16).reshape(n, d2 * 2)


def _mlp_val(terms, adds, b1, w2, b2, g, b):
    acc = jnp.dot(terms[0][0], terms[0][1], preferred_element_type=jnp.float32)
    for xv, wv in terms[1:]:
        acc = acc + jnp.dot(xv, wv, preferred_element_type=jnp.float32)
    for av in adds:
        acc = acc + av
    acc = acc + b1
    hv = acc * lax.logistic(acc)
    yv = jnp.dot(hv, w2, preferred_element_type=jnp.float32) + b2
    if g is not None:
        mu = jnp.mean(yv, axis=-1, keepdims=True)
        var = jnp.mean((yv - mu) ** 2, axis=-1, keepdims=True)
        yv = (yv - mu) * lax.rsqrt(var + 1e-5) * g + b
    return yv


def _tc_mlp(xs, w1s, adds, p, ln=True, res=None, res_is_x0=False,
            extra_out_w=None):
    np_ = (xs + adds)[0].shape[0]
    dh = w1s[0].shape[1] if w1s else adds[0].shape[1]
    dout = p["W2"].shape[1]
    nx, na = len(xs), len(adds)
    has_res = res is not None or res_is_x0
    n_extra = 1 if extra_out_w is not None else 0

    def body(*refs):
        i = 0
        xr = refs[:nx]; i += nx
        ar = refs[i:i + na]; i += na
        wr = refs[i:i + nx]; i += nx
        b1r = refs[i]; w2r = refs[i + 1]; b2r = refs[i + 2]; i += 3
        gr = br = None
        if ln:
            gr, br = refs[i], refs[i + 1]; i += 2
        rr = None
        if res is not None:
            rr = refs[i]; i += 1
        ewr = None
        if n_extra:
            ewr = refs[i]; i += 1
        outr = refs[i]
        out2r = refs[i + 1] if n_extra else None
        y = _mlp_val([(xr[k][...], wr[k][...]) for k in range(nx)],
                     [a[...].astype(jnp.float32) for a in ar], b1r[...],
                     w2r[...], b2r[...], gr[...] if ln else None,
                     br[...] if ln else None)
        if res_is_x0:
            y = y + xr[0][...]
        elif rr is not None:
            y = y + rr[...]
        outr[...] = y
        if n_extra:
            out2r[...] = jnp.dot(
                y, ewr[...],
                preferred_element_type=jnp.float32).astype(jnp.bfloat16)

    grid = (np_ // _RBLK,)
    row = lambda i: (i, 0)
    fix = lambda i: (0, 0)
    in_specs = [pl.BlockSpec((_RBLK, x.shape[1]), row) for x in xs]
    in_specs += [pl.BlockSpec((_RBLK, dh), row) for _ in adds]
    in_specs += [pl.BlockSpec(w.shape, fix) for w in w1s]
    args = list(xs) + list(adds) + list(w1s)
    b1 = p["b1"].reshape(1, dh)
    w2 = p["W2"]
    b2 = p["b2"].reshape(1, dout)
    in_specs += [pl.BlockSpec((1, dh), fix), pl.BlockSpec(w2.shape, fix),
                 pl.BlockSpec((1, dout), fix)]
    args += [b1, w2, b2]
    if ln:
        in_specs += [pl.BlockSpec((1, dout), fix), pl.BlockSpec((1, dout), fix)]
        args += [p["g"].reshape(1, dout), p["b"].reshape(1, dout)]
    if res is not None:
        in_specs += [pl.BlockSpec((_RBLK, dout), row)]
        args += [res]
    out_shape = [jax.ShapeDtypeStruct((np_, dout), jnp.float32)]
    out_specs = [pl.BlockSpec((_RBLK, dout), row)]
    if n_extra:
        in_specs += [pl.BlockSpec(extra_out_w.shape, fix)]
        args += [extra_out_w]
        dk = extra_out_w.shape[1]
        out_shape += [jax.ShapeDtypeStruct((np_, dk), jnp.bfloat16)]
        out_specs += [pl.BlockSpec((_RBLK, dk), row)]
    outs = pl.pallas_call(
        body, grid=grid, in_specs=in_specs, out_specs=out_specs,
        out_shape=out_shape, interpret=_INTERPRET)(*args)
    return outs if n_extra else outs[0]


def _tc_edge_fused(ef, enc_p, gs, gd, edge_p):
    np_ = ef.shape[0]
    din = ef.shape[1]

    def body(efr, gsr, gdr,
             ew1, eb1, ew2, eb2, eg, ebb,
             w1a, b1r, w2r, b2r, gr, br, outr):
        e = _mlp_val([(efr[...], ew1[...])], [], eb1[...], ew2[...], eb2[...],
                     eg[...], ebb[...])
        y = _mlp_val([(e, w1a[...])],
                     [gsr[...].astype(jnp.float32),
                      gdr[...].astype(jnp.float32)],
                     b1r[...], w2r[...], b2r[...], gr[...], br[...])
        outr[...] = e + y

    row = lambda i: (i, 0)
    fix = lambda i: (0, 0)
    in_specs = [pl.BlockSpec((_RBLK, din), row),
                pl.BlockSpec((_RBLK, _HID), row),
                pl.BlockSpec((_RBLK, _HID), row)]
    args = [ef, gs, gd]
    for w, shp in [(enc_p["W1"], None), (enc_p["b1"].reshape(1, _HID), None),
                   (enc_p["W2"], None), (enc_p["b2"].reshape(1, _HID), None),
                   (enc_p["g"].reshape(1, _HID), None),
                   (enc_p["b"].reshape(1, _HID), None),
                   (edge_p["W1"][:_HID], None),
                   (edge_p["b1"].reshape(1, _HID), None),
                   (edge_p["W2"], None), (edge_p["b2"].reshape(1, _HID), None),
                   (edge_p["g"].reshape(1, _HID), None),
                   (edge_p["b"].reshape(1, _HID), None)]:
        in_specs.append(pl.BlockSpec(w.shape, fix))
        args.append(w)
    return pl.pallas_call(
        body, grid=(np_ // _RBLK,), in_specs=in_specs,
        out_specs=pl.BlockSpec((_RBLK, _HID), row),
        out_shape=jax.ShapeDtypeStruct((np_, _HID), jnp.float32),
        interpret=_INTERPRET)(*args)


def _tc_matmul(x, *ws):
    np_, din = x.shape
    nw = len(ws)

    def body(*refs):
        xv = refs[0][...]
        for k in range(nw):
            refs[1 + nw + k][...] = jnp.dot(
                xv, refs[1 + k][...],
                preferred_element_type=jnp.float32).astype(jnp.bfloat16)

    row = lambda i: (i, 0)
    fix = lambda i: (0, 0)
    outs = pl.pallas_call(
        body, grid=(np_ // _RBLK,),
        in_specs=[pl.BlockSpec((_RBLK, din), row)]
                 + [pl.BlockSpec(w.shape, fix) for w in ws],
        out_specs=[pl.BlockSpec((_RBLK, w.shape[1]), row) for w in ws],
        out_shape=[jax.ShapeDtypeStruct((np_, w.shape[1]), jnp.bfloat16)
                   for w in ws],
        interpret=_INTERPRET)(x, *ws)
    return outs if nw > 1 else outs[0]


_NW = 32


def _ds8(start, size):
    return pl.ds(pl.multiple_of(start, 8), size)


def _chunk(bpw, cap=120):
    for c in (120, 80, 40, 16, 8):
        if c <= cap and bpw % c == 0:
            return c
    raise ValueError(bpw)


def _sc_gather2(table_a, idx_a, table_b, idx_b):
    e = idx_a.shape[0]
    d = table_a.shape[1]
    bpw = e // _NW
    cchunk = _chunk(bpw)
    nchunks = bpw // cchunk
    mesh = plsc.VectorSubcoreMesh(core_axis_name="c", subcore_axis_name="s")

    @functools.partial(
        pl.kernel, mesh=mesh,
        out_type=[jax.ShapeDtypeStruct((e, d), jnp.int32)] * 2,
        scratch_types=[pltpu.VMEM((cchunk,), jnp.int32),
                       pltpu.VMEM((cchunk, d), jnp.int32),
                       pltpu.VMEM((cchunk,), jnp.int32),
                       pltpu.VMEM((cchunk, d), jnp.int32),
                       pltpu.SemaphoreType.DMA, pltpu.SemaphoreType.DMA])
    def k(ta, ia, tb, ib, oa, ob, iva, rva, ivb, rvb, sema, semb):
        wid = lax.axis_index("s") * 2 + lax.axis_index("c")

        def step(i, _):
            base = wid * bpw + i * cchunk
            pltpu.sync_copy(ia.at[pl.ds(base, cchunk)], iva)
            pltpu.sync_copy(ib.at[pl.ds(base, cchunk)], ivb)
            ca = pltpu.async_copy(ta.at[iva], rva, sema)
            cb = pltpu.async_copy(tb.at[ivb], rvb, semb)
            ca.wait()
            cb.wait()
            pltpu.sync_copy(rva, oa.at[pl.ds(base, cchunk)])
            pltpu.sync_copy(rvb, ob.at[pl.ds(base, cchunk)])
            return 0

        lax.fori_loop(0, nchunks, step, 0)

    return k(table_a, idx_a, table_b, idx_b)


def _sc_segsum(rows, dst, n_out):
    if n_out * 128 * 4 <= 7 << 20:
        return _sc_segsum_small(rows, dst, n_out)
    return _sc_segsum_rows(rows, dst, n_out)


def _sc_segsum_small(rows, dst, n_out):
    e = rows.shape[0]
    bpw = e // 16
    cchunk = _chunk(bpw)
    nchunks = bpw // cchunk
    ntile_rows = n_out // 16
    assert n_out % 16 == 0
    zeros = jnp.zeros((n_out, 128), jnp.float32)
    mesh = plsc.VectorSubcoreMesh(core_axis_name="c", subcore_axis_name="s")

    @functools.partial(
        pl.kernel, mesh=mesh,
        out_type=jax.ShapeDtypeStruct((2, n_out, 128), jnp.float32),
        scratch_types=[pltpu.VMEM((cchunk,), jnp.int32),
                       pltpu.VMEM((cchunk, 128), jnp.float32),
                       pltpu.VMEM_SHARED((n_out, 128), jnp.float32)])
    def k(rows_h, dst_h, zeros_h, out_h, idx_v, buf_v, acc_s):
        cid = lax.axis_index("c")
        sid = lax.axis_index("s")
        tslice = _ds8(sid * ntile_rows, ntile_rows)

        pltpu.sync_copy(zeros_h.at[tslice], acc_s.at[tslice])
        plsc.subcore_barrier()

        def step(i, _):
            base = sid * bpw + i * cchunk
            pltpu.sync_copy(dst_h.at[_ds8(base, cchunk)], idx_v)
            pltpu.sync_copy(
                rows_h.at[_ds8(base, cchunk), _ds8(cid * 128, 128)], buf_v)
            pltpu.sync_copy(buf_v, acc_s.at[idx_v], add=True)
            return 0

        lax.fori_loop(0, nchunks, step, 0)
        plsc.subcore_barrier()
        pltpu.sync_copy(acc_s.at[tslice], out_h.at[cid, tslice])

    out = k(rows, dst, zeros)
    return out.transpose(1, 0, 2).reshape(n_out, 256)


def _sc_segsum_rows(rows, dst, n_out):
    e = rows.shape[0]
    bpw = e // 16
    cchunk = 80
    assert bpw % cchunk == 0
    nchunks = bpw // cchunk
    rp = 13184
    npass = -(-n_out // rp)
    acc_rows = rp + 128
    zeros = jnp.zeros((acc_rows, 128), jnp.float32)
    mesh = plsc.VectorSubcoreMesh(core_axis_name="c", subcore_axis_name="s")

    @functools.partial(
        pl.kernel, mesh=mesh,
        out_type=jax.ShapeDtypeStruct((2, npass * rp, 128), jnp.float32),
        scratch_types=[pltpu.VMEM((cchunk,), jnp.int32),
                       pltpu.VMEM((cchunk,), jnp.int32),
                       pltpu.VMEM((cchunk, 128), jnp.float32),
                       pltpu.VMEM_SHARED((acc_rows, 128), jnp.float32)])
    def k(rows_h, dst_h, zeros_h, out_h, idx_v, idx2_v, buf_v, acc_s):
        cid = lax.axis_index("c")
        sid = lax.axis_index("s")
        zslice = _ds8(sid * (acc_rows // 16), acc_rows // 16)

        for pp in range(npass):
            seg0 = pp * rp
            pltpu.sync_copy(zeros_h.at[zslice], acc_s.at[zslice])
            plsc.subcore_barrier()

            def step(i, _, seg0=seg0):
                base = sid * bpw + i * cchunk
                pltpu.sync_copy(dst_h.at[_ds8(base, cchunk)], idx_v)
                for j in range(cchunk // 16):
                    v = idx_v[pl.ds(j * 16, 16)]
                    local = v - seg0
                    ok = (local >= 0) & (local < rp)
                    idx2_v[pl.ds(j * 16, 16)] = jnp.where(ok, local, rp)
                pltpu.sync_copy(
                    rows_h.at[_ds8(base, cchunk), _ds8(cid * 128, 128)],
                    buf_v)
                pltpu.sync_copy(buf_v, acc_s.at[idx2_v], add=True)
                return 0

            lax.fori_loop(0, nchunks, step, 0)
            plsc.subcore_barrier()
            pltpu.sync_copy(acc_s.at[_ds8(sid * (rp // 16), rp // 16)],
                            out_h.at[cid, _ds8(seg0 + sid * (rp // 16),
                                               rp // 16)])
            plsc.subcore_barrier()

    out = k(rows, dst, zeros)
    return out.transpose(1, 0, 2).reshape(npass * rp, 256)[:n_out]


_USE_SC_GATHER = True
_USE_SC_SEGSUM = True


def _gather2(ta, ia, tb, ib):
    ta, tb = _pack_i32(ta), _pack_i32(tb)
    if _USE_SC_GATHER:
        ga, gb = _sc_gather2(ta, ia, tb, ib)
    else:
        ga, gb = ta[ia], tb[ib]
    return _unpack_bf16(ga), _unpack_bf16(gb)


def _segsum(rows, dst, n_out):
    if _USE_SC_SEGSUM:
        return _sc_segsum(rows, dst, n_out)
    return jax.ops.segment_sum(rows, dst, num_segments=n_out)


def kernel(x, edge_g2m, edge_mesh, edge_m2g, params):
    p = params
    grid_in = _pad_rows(x.reshape(_C_IN, _N_GRID).T, _NP_GRID)
    mesh_in = _pad_rows(_pad_cols(p["mesh_nfeat"], 8), _NP_MESH)

    src_g, dst_g = edge_g2m[0], edge_g2m[1]
    src_g = jnp.pad(src_g, (0, _NP_EG - src_g.shape[0]))
    dst_g = jnp.pad(dst_g, (0, _NP_EG - dst_g.shape[0]),
                    constant_values=_N_MESH)
    ms, md = edge_mesh[0], edge_mesh[1]
    ms = jnp.pad(ms, (0, _NP_EM - ms.shape[0]))
    md = jnp.pad(md, (0, _NP_EM - md.shape[0]), constant_values=_N_MESH)
    s3, d3 = edge_m2g[0], edge_m2g[1]
    s3 = jnp.pad(s3, (0, _NP_EG - s3.shape[0]))
    d3 = jnp.pad(d3, (0, _NP_EG - d3.shape[0]), constant_values=_N_GRID)

    ef_g2m = _pad_rows(_pad_cols(p["efeat_g2m"], 8), _NP_EG)
    ef_mesh = _pad_rows(_pad_cols(p["efeat_mesh"], 8), _NP_EM)
    ef_m2g = _pad_rows(_pad_cols(p["efeat_m2g"], 8), _NP_EG)

    enc_grid = dict(p["enc_grid"])
    enc_mesh = dict(p["enc_mesh"])
    enc_mesh = {**enc_mesh, "W1": jnp.pad(enc_mesh["W1"], ((0, 5), (0, 0)))}
    enc_eg2m = {**p["enc_eg2m"],
                "W1": jnp.pad(p["enc_eg2m"]["W1"], ((0, 4), (0, 0)))}
    enc_emesh = {**p["enc_emesh"],
                 "W1": jnp.pad(p["enc_emesh"]["W1"], ((0, 4), (0, 0)))}
    enc_em2g = {**p["enc_em2g"],
                "W1": jnp.pad(p["enc_em2g"]["W1"], ((0, 4), (0, 0)))}

    w_g2m = p["g2m_edge"]["W1"]
    gfeat, ts = _tc_mlp([grid_in], [enc_grid["W1"]], [], enc_grid,
                        extra_out_w=w_g2m[_HID:2 * _HID])
    mfeat, td = _tc_mlp([mesh_in], [enc_mesh["W1"]], [], enc_mesh,
                        extra_out_w=w_g2m[2 * _HID:])
    e2 = _tc_mlp([ef_mesh], [enc_emesh["W1"]], [], enc_emesh)

    gs, gd = _gather2(ts, src_g, td, dst_g)
    e1 = _tc_edge_fused(ef_g2m, enc_eg2m, gs, gd, p["g2m_edge"])
    agg = _segsum(e1, dst_g, _NP_MESH)
    wn = p["g2m_node"]["W1"]
    mfeat = _tc_mlp([mfeat, agg], [wn[:_HID], wn[_HID:]], [], p["g2m_node"],
                    res_is_x0=True)
    gfeat = _tc_mlp([gfeat], [p["g2m_grid"]["W1"]], [], p["g2m_grid"],
                    res_is_x0=True)

    for lp in p["proc"]:
        w1 = lp["edge"]["W1"]
        ts, td = _tc_matmul(mfeat, w1[_HID:2 * _HID], w1[2 * _HID:])
        gs, gd = _gather2(ts, ms, td, md)
        e2 = _tc_mlp([e2], [w1[:_HID]], [gs, gd], lp["edge"], res_is_x0=True)
        agg = _segsum(e2, md, _NP_MESH)
        wn = lp["node"]["W1"]
        mfeat = _tc_mlp([mfeat, agg], [wn[:_HID], wn[_HID:]], [], lp["node"],
                        res_is_x0=True)

    w1 = p["m2g_edge"]["W1"]
    ts = _tc_matmul(mfeat, w1[_HID:2 * _HID])
    td = _tc_matmul(gfeat, w1[2 * _HID:])
    gs, gd = _gather2(ts, s3, td, d3)
    e3 = _tc_edge_fused(ef_m2g, enc_em2g, gs, gd, p["m2g_edge"])
    agg = _segsum(e3, d3, _NP_GRID)
    wn = p["m2g_node"]["W1"]
    gfeat = _tc_mlp([gfeat, agg], [wn[:_HID], wn[_HID:]], [], p["m2g_node"],
                    res_is_x0=True)
    out = _tc_mlp([gfeat], [p["dec_out"]["W1"]], [], p["dec_out"], ln=False)
    return out[:_N_GRID].T.reshape(1, _C_OUT, _H, _W)


# speedup vs baseline: 2.4807x; 2.4807x over previous
import functools

import jax
import jax.numpy as jnp
from jax import lax
from jax.experimental import pallas as pl
from jax.experimental.pallas import tpu as pltpu
from jax.experimental.pallas import tpu_sc as plsc

_H, _W = 181, 360
_N_GRID = _H * _W
_N_MESH = 10242
_C_IN = 128
_C_OUT = 128
_HID = 256
_L = 4

_NP_GRID = 65536
_NP_MESH = 10752
_NP_EG = 130560
_NP_EM = 40960

_RBLK = 512
_INTERPRET = False
_USE_SC = True


def _pad_rows(a, n):
    return jnp.pad(a, ((0, n - a.shape[0]), (0, 0)))


def _pad_cols(a, n):
    return jnp.pad(a, ((0, 0), (0, n - a.shape[1])))


def _mlp_val(terms, adds, b1, w2, b2, g, b):
    terms = [(x.astype(jnp.float32), w) for x, w in terms]
    acc = jnp.dot(terms[0][0], terms[0][1], preferred_element_type=jnp.float32)
    for xv, wv in terms[1:]:
        acc = acc + jnp.dot(xv, wv, preferred_element_type=jnp.float32)
    for av in adds:
        acc = acc + av
    acc = acc + b1
    hv = acc * lax.logistic(acc)
    yv = jnp.dot(hv, w2, preferred_element_type=jnp.float32) + b2
    if g is not None:
        mu = jnp.mean(yv, axis=-1, keepdims=True)
        var = jnp.mean((yv - mu) ** 2, axis=-1, keepdims=True)
        yv = (yv - mu) * lax.rsqrt(var + 1e-5) * g + b
    return yv


def _tc_mlp(xs, w1s, adds, p, ln=True, res=None, res_is_x0=False,
            extra_out_w=None):
    np_ = (xs + adds)[0].shape[0]
    dh = w1s[0].shape[1] if w1s else adds[0].shape[1]
    dout = p["W2"].shape[1]
    nx, na = len(xs), len(adds)
    has_res = res is not None or res_is_x0
    n_extra = 1 if extra_out_w is not None else 0

    def body(*refs):
        i = 0
        xr = refs[:nx]; i += nx
        ar = refs[i:i + na]; i += na
        wr = refs[i:i + nx]; i += nx
        b1r = refs[i]; w2r = refs[i + 1]; b2r = refs[i + 2]; i += 3
        gr = br = None
        if ln:
            gr, br = refs[i], refs[i + 1]; i += 2
        rr = None
        if res is not None:
            rr = refs[i]; i += 1
        ewr = None
        if extra_out_w is not None:
            ewr = refs[i]; i += 1
        outr = refs[i]
        out2r = refs[i + 1] if n_extra else None
        y = _mlp_val([(xr[k][...], wr[k][...]) for k in range(nx)],
                     [a[...].astype(jnp.float32) for a in ar], b1r[...],
                     w2r[...], b2r[...], gr[...] if ln else None,
                     br[...] if ln else None)
        if res_is_x0:
            y = y + xr[0][...]
        elif rr is not None:
            y = y + rr[...]
        outr[...] = y
        if extra_out_w is not None:
            out2r[...] = jnp.dot(y, ewr[...],
                                 preferred_element_type=jnp.float32)

    grid = (np_ // _RBLK,)
    row = lambda i: (i, 0)
    fix = lambda i: (0, 0)
    in_specs = [pl.BlockSpec((_RBLK, x.shape[1]), row) for x in xs]
    in_specs += [pl.BlockSpec((_RBLK, dh), row) for _ in adds]
    in_specs += [pl.BlockSpec(w.shape, fix) for w in w1s]
    args = list(xs) + list(adds) + list(w1s)
    b1 = p["b1"].reshape(1, dh)
    w2 = p["W2"]
    b2 = p["b2"].reshape(1, dout)
    in_specs += [pl.BlockSpec((1, dh), fix), pl.BlockSpec(w2.shape, fix),
                 pl.BlockSpec((1, dout), fix)]
    args += [b1, w2, b2]
    if ln:
        in_specs += [pl.BlockSpec((1, dout), fix), pl.BlockSpec((1, dout), fix)]
        args += [p["g"].reshape(1, dout), p["b"].reshape(1, dout)]
    if res is not None:
        in_specs += [pl.BlockSpec((_RBLK, dout), row)]
        args += [res]
    out_shape = [jax.ShapeDtypeStruct((np_, dout), jnp.float32)]
    out_specs = [pl.BlockSpec((_RBLK, dout), row)]
    if extra_out_w is not None:
        in_specs += [pl.BlockSpec(extra_out_w.shape, fix)]
        args += [extra_out_w]
        dk = extra_out_w.shape[1]
        out_shape += [jax.ShapeDtypeStruct((np_, dk), jnp.float32)]
        out_specs += [pl.BlockSpec((_RBLK, dk), row)]
    outs = pl.pallas_call(
        body, grid=grid, in_specs=in_specs, out_specs=out_specs,
        out_shape=out_shape, interpret=_INTERPRET)(*args)
    return outs if n_extra else outs[0]


def _tc_edge_fused(ef, enc_p, gs, gd, edge_p):
    np_ = ef.shape[0]
    din = ef.shape[1]

    def body(efr, gsr, gdr,
             ew1, eb1, ew2, eb2, eg, ebb,
             w1a, b1r, w2r, b2r, gr, br, outr):
        e = _mlp_val([(efr[...], ew1[...])], [], eb1[...], ew2[...], eb2[...],
                     eg[...], ebb[...])
        y = _mlp_val([(e, w1a[...])],
                     [gsr[...].astype(jnp.float32),
                      gdr[...].astype(jnp.float32)],
                     b1r[...], w2r[...], b2r[...], gr[...], br[...])
        outr[...] = e + y

    row = lambda i: (i, 0)
    fix = lambda i: (0, 0)
    in_specs = [pl.BlockSpec((_RBLK, din), row),
                pl.BlockSpec((_RBLK, _HID), row),
                pl.BlockSpec((_RBLK, _HID), row)]
    args = [ef, gs, gd]
    for w, shp in [(enc_p["W1"], None), (enc_p["b1"].reshape(1, _HID), None),
                   (enc_p["W2"], None), (enc_p["b2"].reshape(1, _HID), None),
                   (enc_p["g"].reshape(1, _HID), None),
                   (enc_p["b"].reshape(1, _HID), None),
                   (edge_p["W1"][:_HID], None),
                   (edge_p["b1"].reshape(1, _HID), None),
                   (edge_p["W2"], None), (edge_p["b2"].reshape(1, _HID), None),
                   (edge_p["g"].reshape(1, _HID), None),
                   (edge_p["b"].reshape(1, _HID), None)]:
        in_specs.append(pl.BlockSpec(w.shape, fix))
        args.append(w)
    return pl.pallas_call(
        body, grid=(np_ // _RBLK,), in_specs=in_specs,
        out_specs=pl.BlockSpec((_RBLK, _HID), row),
        out_shape=jax.ShapeDtypeStruct((np_, _HID), jnp.float32),
        interpret=_INTERPRET)(*args)


def _tc_matmul(x, *ws):
    np_, din = x.shape
    nw = len(ws)

    def body(*refs):
        xv = refs[0][...]
        for k in range(nw):
            refs[1 + nw + k][...] = jnp.dot(
                xv, refs[1 + k][...], preferred_element_type=jnp.float32)

    row = lambda i: (i, 0)
    fix = lambda i: (0, 0)
    outs = pl.pallas_call(
        body, grid=(np_ // _RBLK,),
        in_specs=[pl.BlockSpec((_RBLK, din), row)]
                 + [pl.BlockSpec(w.shape, fix) for w in ws],
        out_specs=[pl.BlockSpec((_RBLK, w.shape[1]), row) for w in ws],
        out_shape=[jax.ShapeDtypeStruct((np_, w.shape[1]), jnp.float32)
                   for w in ws],
        interpret=_INTERPRET)(x, *ws)
    return outs if nw > 1 else outs[0]


_NW = 32


def _ds8(start, size, align=8):
    return pl.ds(pl.multiple_of(start, align), size)


def _chunk(bpw, cap=80, even=False):
    for c in (80, 48, 40, 16, 8):
        if c <= cap and bpw % c == 0 and (not even or (bpw // c) % 2 == 0):
            return c
    raise ValueError(bpw)


def _sc_gather2(table_a, idx_a, table_b, idx_b):
    e = idx_a.shape[0]
    d = table_a.shape[1]
    bpw = e // _NW
    cchunk = _chunk(bpw, even=True)
    nchunks = bpw // cchunk
    mesh = plsc.VectorSubcoreMesh(core_axis_name="c", subcore_axis_name="s")
    dt = table_a.dtype

    @functools.partial(
        pl.kernel, mesh=mesh,
        out_type=[jax.ShapeDtypeStruct((e, d), dt)] * 2,
        scratch_types=[pltpu.VMEM((cchunk,), jnp.int32),
                       pltpu.VMEM((cchunk,), jnp.int32),
                       pltpu.VMEM((cchunk,), jnp.int32),
                       pltpu.VMEM((cchunk,), jnp.int32),
                       pltpu.VMEM((cchunk, d), dt),
                       pltpu.VMEM((cchunk, d), dt),
                       pltpu.VMEM((cchunk, d), dt),
                       pltpu.VMEM((cchunk, d), dt)]
                      + [pltpu.SemaphoreType.DMA] * 8)
    def k(ta, ia, tb, ib, oa, ob, iva0, iva1, ivb0, ivb1,
          ra0, ra1, rb0, rb1,
          g0a, g0b, g1a, g1b, w0a, w0b, w1a, w1b):
        wid = lax.axis_index("s") * 2 + lax.axis_index("c")
        ivas, ivbs = (iva0, iva1), (ivb0, ivb1)
        ras, rbs = (ra0, ra1), (rb0, rb1)
        gsems = ((g0a, g0b), (g1a, g1b))
        wsems = ((w0a, w0b), (w1a, w1b))

        def start_gather(c, bi):
            base = wid * bpw + c * cchunk
            pltpu.sync_copy(ia.at[_ds8(base, cchunk)], ivas[bi])
            pltpu.sync_copy(ib.at[_ds8(base, cchunk)], ivbs[bi])
            pltpu.async_copy(ta.at[ivas[bi]], ras[bi], gsems[bi][0])
            pltpu.async_copy(tb.at[ivbs[bi]], rbs[bi], gsems[bi][1])

        def wait_gather(bi):
            pltpu.make_async_copy(ta.at[ivas[bi]], ras[bi],
                                  gsems[bi][0]).wait()
            pltpu.make_async_copy(tb.at[ivbs[bi]], rbs[bi],
                                  gsems[bi][1]).wait()

        def start_write(c, bi):
            base = wid * bpw + c * cchunk
            pltpu.async_copy(ras[bi], oa.at[_ds8(base, cchunk)], wsems[bi][0])
            pltpu.async_copy(rbs[bi], ob.at[_ds8(base, cchunk)], wsems[bi][1])

        def wait_write(bi):
            pltpu.make_async_copy(ras[bi], oa.at[pl.ds(0, cchunk)],
                                  wsems[bi][0]).wait()
            pltpu.make_async_copy(rbs[bi], ob.at[pl.ds(0, cchunk)],
                                  wsems[bi][1]).wait()

        start_gather(0, 0)

        def body(i2, _):
            c0 = 2 * i2
            c1 = c0 + 1
            c2 = c0 + 2

            @pl.when(i2 > 0)
            def _():
                wait_write(1)

            start_gather(c1, 1)
            wait_gather(0)
            start_write(c0, 0)
            wait_gather(1)
            start_write(c1, 1)

            @pl.when(c2 < nchunks)
            def _():
                wait_write(0)
                start_gather(c2, 0)

            return 0

        lax.fori_loop(0, nchunks // 2, body, 0)
        wait_write(0)
        wait_write(1)

    return k(table_a, idx_a, table_b, idx_b)


def _sc_segsum(rows, dst, n_out):
    if n_out * 128 * 4 <= 7 << 20:
        return _sc_segsum_small(rows, dst, n_out)
    return _sc_segsum_rows(rows, dst, n_out)


def _sc_segsum_small(rows, dst, n_out):
    e = rows.shape[0]
    dt = rows.dtype
    al = 16 if dt == jnp.bfloat16 else 8
    bpw = e // 16
    cchunk = _chunk(bpw, even=True)
    nchunks = bpw // cchunk
    ntile_rows = n_out // 16
    assert n_out % 16 == 0
    zeros = jnp.zeros((n_out, 128), dt)
    mesh = plsc.VectorSubcoreMesh(core_axis_name="c", subcore_axis_name="s")

    @functools.partial(
        pl.kernel, mesh=mesh,
        out_type=jax.ShapeDtypeStruct((2, n_out, 128), dt),
        scratch_types=[pltpu.VMEM((cchunk,), jnp.int32),
                       pltpu.VMEM((cchunk,), jnp.int32),
                       pltpu.VMEM((cchunk, 128), dt),
                       pltpu.VMEM((cchunk, 128), dt),
                       pltpu.VMEM_SHARED((n_out, 128), dt)]
                      + [pltpu.SemaphoreType.DMA] * 4)
    def k(rows_h, dst_h, zeros_h, out_h, iv0, iv1, b0, b1, acc_s,
          si0, si1, sr0, sr1):
        cid = lax.axis_index("c")
        sid = lax.axis_index("s")
        ivs, bufs = (iv0, iv1), (b0, b1)
        isems, rsems = (si0, si1), (sr0, sr1)
        tslice = _ds8(sid * ntile_rows, ntile_rows, al)

        pltpu.sync_copy(zeros_h.at[tslice], acc_s.at[tslice])
        plsc.subcore_barrier()

        def start_dma(c, bi):
            base = sid * bpw + c * cchunk
            pltpu.async_copy(dst_h.at[_ds8(base, cchunk)], ivs[bi], isems[bi])
            pltpu.async_copy(
                rows_h.at[_ds8(base, cchunk, al), _ds8(cid * 128, 128)],
                bufs[bi], rsems[bi])

        def wait_dma(bi):
            pltpu.make_async_copy(dst_h.at[pl.ds(0, cchunk)], ivs[bi],
                                  isems[bi]).wait()
            pltpu.make_async_copy(rows_h.at[pl.ds(0, cchunk),
                                            pl.ds(0, 128)],
                                  bufs[bi], rsems[bi]).wait()

        start_dma(0, 0)

        def body(i2, _):
            c2 = 2 * i2 + 2
            start_dma(2 * i2 + 1, 1)
            wait_dma(0)
            pltpu.sync_copy(b0, acc_s.at[iv0], add=True)

            @pl.when(c2 < nchunks)
            def _():
                start_dma(c2, 0)

            wait_dma(1)
            pltpu.sync_copy(b1, acc_s.at[iv1], add=True)
            return 0

        lax.fori_loop(0, nchunks // 2, body, 0)
        plsc.subcore_barrier()
        pltpu.sync_copy(acc_s.at[tslice], out_h.at[cid, tslice])

    out = k(rows, dst, zeros)
    return out.transpose(1, 0, 2).reshape(n_out, 256)


def _sc_segsum_rows(rows, dst, n_out):
    e = rows.shape[0]
    dt = rows.dtype
    al = 16 if dt == jnp.bfloat16 else 8
    bpw = e // 16
    cchunk = 80
    assert bpw % cchunk == 0 and (bpw // cchunk) % 2 == 0
    nchunks = bpw // cchunk
    rp = 26368 if dt == jnp.bfloat16 else 13184
    npass = -(-n_out // rp)
    acc_rows = rp + 256
    zeros = jnp.zeros((acc_rows, 128), dt)
    mesh = plsc.VectorSubcoreMesh(core_axis_name="c", subcore_axis_name="s")

    @functools.partial(
        pl.kernel, mesh=mesh,
        out_type=jax.ShapeDtypeStruct((2, npass * rp, 128), dt),
        scratch_types=[pltpu.VMEM((cchunk,), jnp.int32),
                       pltpu.VMEM((cchunk,), jnp.int32),
                       pltpu.VMEM((cchunk,), jnp.int32),
                       pltpu.VMEM((cchunk,), jnp.int32),
                       pltpu.VMEM((cchunk, 128), dt),
                       pltpu.VMEM((cchunk, 128), dt),
                       pltpu.VMEM_SHARED((acc_rows, 128), dt)]
                      + [pltpu.SemaphoreType.DMA] * 4)
    def k(rows_h, dst_h, zeros_h, out_h, iv0, iv1, ix0, ix1, b0, b1, acc_s,
          si0, si1, sr0, sr1):
        cid = lax.axis_index("c")
        sid = lax.axis_index("s")
        ivs, ixs, bufs = (iv0, iv1), (ix0, ix1), (b0, b1)
        isems, rsems = (si0, si1), (sr0, sr1)
        zslice = _ds8(sid * (acc_rows // 16), acc_rows // 16, al)

        def start_dma(c, bi):
            base = sid * bpw + c * cchunk
            pltpu.async_copy(dst_h.at[_ds8(base, cchunk)], ivs[bi], isems[bi])
            pltpu.async_copy(
                rows_h.at[_ds8(base, cchunk, al), _ds8(cid * 128, 128)],
                bufs[bi], rsems[bi])

        def wait_dma(bi):
            pltpu.make_async_copy(dst_h.at[pl.ds(0, cchunk)], ivs[bi],
                                  isems[bi]).wait()
            pltpu.make_async_copy(rows_h.at[pl.ds(0, cchunk), pl.ds(0, 128)],
                                  bufs[bi], rsems[bi]).wait()

        def rebase_scatter(bi, seg0):
            for j in range(cchunk // 16):
                v = ivs[bi][pl.ds(j * 16, 16)]
                local = v - seg0
                ok = (local >= 0) & (local < rp)
                ixs[bi][pl.ds(j * 16, 16)] = jnp.where(ok, local, rp)
            pltpu.sync_copy(bufs[bi], acc_s.at[ixs[bi]], add=True)

        for pp in range(npass):
            seg0 = pp * rp
            pltpu.sync_copy(zeros_h.at[zslice], acc_s.at[zslice])
            plsc.subcore_barrier()
            start_dma(0, 0)

            def body(i2, _, seg0=seg0):
                c2 = 2 * i2 + 2
                start_dma(2 * i2 + 1, 1)
                wait_dma(0)
                rebase_scatter(0, seg0)

                @pl.when(c2 < nchunks)
                def _():
                    start_dma(c2, 0)

                wait_dma(1)
                rebase_scatter(1, seg0)
                return 0

            lax.fori_loop(0, nchunks // 2, body, 0)
            plsc.subcore_barrier()
            pltpu.sync_copy(acc_s.at[_ds8(sid * (rp // 16), rp // 16, al)],
                            out_h.at[cid, _ds8(seg0 + sid * (rp // 16),
                                               rp // 16, al)])
            plsc.subcore_barrier()

    out = k(rows, dst, zeros)
    return out.transpose(1, 0, 2).reshape(npass * rp, 256)[:n_out]


_USE_SC_GATHER = True
_USE_SC_SEGSUM = True


def _gather2(ta, ia, tb, ib):
    if _USE_SC_GATHER:
        return _sc_gather2(ta, ia, tb, ib)
    return ta[ia], tb[ib]


def _segsum(rows, dst, n_out):
    if _USE_SC_SEGSUM:
        return _sc_segsum(rows, dst, n_out)
    return jax.ops.segment_sum(rows, dst, num_segments=n_out)


def kernel(x, edge_g2m, edge_mesh, edge_m2g, params):
    p = params
    grid_in = _pad_rows(x.reshape(_C_IN, _N_GRID).T, _NP_GRID)
    mesh_in = _pad_rows(_pad_cols(p["mesh_nfeat"], 8), _NP_MESH)

    src_g, dst_g = edge_g2m[0], edge_g2m[1]
    src_g = jnp.pad(src_g, (0, _NP_EG - src_g.shape[0]))
    dst_g = jnp.pad(dst_g, (0, _NP_EG - dst_g.shape[0]),
                    constant_values=_N_MESH)
    ms, md = edge_mesh[0], edge_mesh[1]
    ms = jnp.pad(ms, (0, _NP_EM - ms.shape[0]))
    md = jnp.pad(md, (0, _NP_EM - md.shape[0]), constant_values=_N_MESH)
    s3, d3 = edge_m2g[0], edge_m2g[1]
    s3 = jnp.pad(s3, (0, _NP_EG - s3.shape[0]))
    d3 = jnp.pad(d3, (0, _NP_EG - d3.shape[0]), constant_values=_N_GRID)

    ef_g2m = _pad_rows(_pad_cols(p["efeat_g2m"], 8), _NP_EG)
    ef_mesh = _pad_rows(_pad_cols(p["efeat_mesh"], 8), _NP_EM)
    ef_m2g = _pad_rows(_pad_cols(p["efeat_m2g"], 8), _NP_EG)

    enc_grid = dict(p["enc_grid"])
    enc_mesh = dict(p["enc_mesh"])
    enc_mesh = {**enc_mesh, "W1": jnp.pad(enc_mesh["W1"], ((0, 5), (0, 0)))}
    enc_eg2m = {**p["enc_eg2m"],
                "W1": jnp.pad(p["enc_eg2m"]["W1"], ((0, 4), (0, 0)))}
    enc_emesh = {**p["enc_emesh"],
                 "W1": jnp.pad(p["enc_emesh"]["W1"], ((0, 4), (0, 0)))}
    enc_em2g = {**p["enc_em2g"],
                "W1": jnp.pad(p["enc_em2g"]["W1"], ((0, 4), (0, 0)))}

    w_g2m = p["g2m_edge"]["W1"]
    gfeat, ts = _tc_mlp([grid_in], [enc_grid["W1"]], [], enc_grid,
                        extra_out_w=w_g2m[_HID:2 * _HID])
    mfeat, td = _tc_mlp([mesh_in], [enc_mesh["W1"]], [], enc_mesh,
                        extra_out_w=w_g2m[2 * _HID:])
    e2 = _tc_mlp([ef_mesh], [enc_emesh["W1"]], [], enc_emesh)

    gs, gd = _gather2(ts, src_g, td, dst_g)
    e1 = _tc_edge_fused(ef_g2m, enc_eg2m, gs, gd, p["g2m_edge"])
    agg = _segsum(e1, dst_g, _NP_MESH)
    wn = p["g2m_node"]["W1"]
    mfeat = _tc_mlp([mfeat, agg], [wn[:_HID], wn[_HID:]], [], p["g2m_node"],
                    res_is_x0=True)
    gfeat = _tc_mlp([gfeat], [p["g2m_grid"]["W1"]], [], p["g2m_grid"],
                    res_is_x0=True)

    for lp in p["proc"]:
        w1 = lp["edge"]["W1"]
        ts, td = _tc_matmul(mfeat, w1[_HID:2 * _HID], w1[2 * _HID:])
        gs, gd = _gather2(ts, ms, td, md)
        e2 = _tc_mlp([e2], [w1[:_HID]], [gs, gd], lp["edge"],
                     res_is_x0=True)
        agg = _segsum(e2, md, _NP_MESH)
        wn = lp["node"]["W1"]
        mfeat = _tc_mlp([mfeat, agg], [wn[:_HID], wn[_HID:]], [], lp["node"],
                        res_is_x0=True)

    w1 = p["m2g_edge"]["W1"]
    ts = _tc_matmul(mfeat, w1[_HID:2 * _HID])
    td = _tc_matmul(gfeat, w1[2 * _HID:])
    gs, gd = _gather2(ts, s3, td, d3)
    e3 = _tc_edge_fused(ef_m2g, enc_em2g, gs, gd, p["m2g_edge"])
    agg = _segsum(e3, d3, _NP_GRID)
    wn = p["m2g_node"]["W1"]
    gfeat = _tc_mlp([gfeat, agg], [wn[:_HID], wn[_HID:]], [], p["m2g_node"],
                    res_is_x0=True)
    out = _tc_mlp([gfeat], [p["dec_out"]["W1"]], [], p["dec_out"], ln=False)
    return out[:_N_GRID].T.reshape(1, _C_OUT, _H, _W)


# speedup vs baseline: 2.6633x; 1.0736x over previous
import functools

import jax
import jax.numpy as jnp
from jax import lax
from jax.experimental import pallas as pl
from jax.experimental.pallas import tpu as pltpu
from jax.experimental.pallas import tpu_sc as plsc

_H, _W = 181, 360
_N_GRID = _H * _W
_N_MESH = 10242
_C_IN = 128
_C_OUT = 128
_HID = 256
_L = 4

_NP_GRID = 65536
_NP_MESH = 10752
_NP_EG = 130560
_NP_EM = 40960

_RBLK = 512
_INTERPRET = False
_USE_SC = True


def _pad_rows(a, n):
    return jnp.pad(a, ((0, n - a.shape[0]), (0, 0)))


def _pad_cols(a, n):
    return jnp.pad(a, ((0, 0), (0, n - a.shape[1])))


def _pack_val(t):
    d = t.shape[1] // 2
    a = lax.bitcast_convert_type(t[:, :d], jnp.int32)
    b = lax.bitcast_convert_type(t[:, d:], jnp.int32)
    a = ((a + 0x8000) >> 16) & 0xFFFF
    b = (b + 0x8000) & jnp.int32(-65536)
    return a | b


def _unpack_val(g):
    lo = lax.bitcast_convert_type(g << 16, jnp.float32)
    hi = lax.bitcast_convert_type(g & jnp.int32(-65536), jnp.float32)
    return jnp.concatenate([lo, hi], axis=1)


def _mlp_val(terms, adds, b1, w2, b2, g, b):
    terms = [(x.astype(jnp.float32), w) for x, w in terms]
    acc = jnp.dot(terms[0][0], terms[0][1], preferred_element_type=jnp.float32)
    for xv, wv in terms[1:]:
        acc = acc + jnp.dot(xv, wv, preferred_element_type=jnp.float32)
    for av in adds:
        acc = acc + av
    acc = acc + b1
    hv = acc * lax.logistic(acc)
    yv = jnp.dot(hv, w2, preferred_element_type=jnp.float32) + b2
    if g is not None:
        mu = jnp.mean(yv, axis=-1, keepdims=True)
        var = jnp.mean((yv - mu) ** 2, axis=-1, keepdims=True)
        yv = (yv - mu) * lax.rsqrt(var + 1e-5) * g + b
    return yv


def _tc_mlp(xs, w1s, adds, p, ln=True, res=None, res_is_x0=False,
            extra_out_w=None):
    np_ = (xs + adds)[0].shape[0]
    dh = w1s[0].shape[1] if w1s else adds[0].shape[1]
    dout = p["W2"].shape[1]
    nx, na = len(xs), len(adds)
    has_res = res is not None or res_is_x0
    n_extra = 1 if extra_out_w is not None else 0

    def body(*refs):
        i = 0
        xr = refs[:nx]; i += nx
        ar = refs[i:i + na]; i += na
        wr = refs[i:i + nx]; i += nx
        b1r = refs[i]; w2r = refs[i + 1]; b2r = refs[i + 2]; i += 3
        gr = br = None
        if ln:
            gr, br = refs[i], refs[i + 1]; i += 2
        rr = None
        if res is not None:
            rr = refs[i]; i += 1
        ewr = None
        if extra_out_w is not None:
            ewr = refs[i]; i += 1
        outr = refs[i]
        out2r = refs[i + 1] if n_extra else None
        y = _mlp_val([(xr[k][...], wr[k][...]) for k in range(nx)],
                     [_unpack_val(a[...]) for a in ar], b1r[...],
                     w2r[...], b2r[...], gr[...] if ln else None,
                     br[...] if ln else None)
        if res_is_x0:
            y = y + xr[0][...]
        elif rr is not None:
            y = y + rr[...]
        outr[...] = y
        if extra_out_w is not None:
            out2r[...] = _pack_val(jnp.dot(
                y, ewr[...], preferred_element_type=jnp.float32))

    grid = (np_ // _RBLK,)
    row = lambda i: (i, 0)
    fix = lambda i: (0, 0)
    in_specs = [pl.BlockSpec((_RBLK, x.shape[1]), row) for x in xs]
    in_specs += [pl.BlockSpec((_RBLK, dh // 2), row) for _ in adds]
    in_specs += [pl.BlockSpec(w.shape, fix) for w in w1s]
    args = list(xs) + list(adds) + list(w1s)
    b1 = p["b1"].reshape(1, dh)
    w2 = p["W2"]
    b2 = p["b2"].reshape(1, dout)
    in_specs += [pl.BlockSpec((1, dh), fix), pl.BlockSpec(w2.shape, fix),
                 pl.BlockSpec((1, dout), fix)]
    args += [b1, w2, b2]
    if ln:
        in_specs += [pl.BlockSpec((1, dout), fix), pl.BlockSpec((1, dout), fix)]
        args += [p["g"].reshape(1, dout), p["b"].reshape(1, dout)]
    if res is not None:
        in_specs += [pl.BlockSpec((_RBLK, dout), row)]
        args += [res]
    out_shape = [jax.ShapeDtypeStruct((np_, dout), jnp.float32)]
    out_specs = [pl.BlockSpec((_RBLK, dout), row)]
    if extra_out_w is not None:
        in_specs += [pl.BlockSpec(extra_out_w.shape, fix)]
        args += [extra_out_w]
        dk = extra_out_w.shape[1] // 2
        out_shape += [jax.ShapeDtypeStruct((np_, dk), jnp.int32)]
        out_specs += [pl.BlockSpec((_RBLK, dk), row)]
    outs = pl.pallas_call(
        body, grid=grid, in_specs=in_specs, out_specs=out_specs,
        out_shape=out_shape, interpret=_INTERPRET)(*args)
    return outs if n_extra else outs[0]


def _tc_edge_fused(ef, enc_p, gs, gd, edge_p):
    np_ = ef.shape[0]
    din = ef.shape[1]

    def body(efr, gsr, gdr,
             ew1, eb1, ew2, eb2, eg, ebb,
             w1a, b1r, w2r, b2r, gr, br, outr):
        e = _mlp_val([(efr[...], ew1[...])], [], eb1[...], ew2[...], eb2[...],
                     eg[...], ebb[...])
        y = _mlp_val([(e, w1a[...])],
                     [_unpack_val(gsr[...]), _unpack_val(gdr[...])],
                     b1r[...], w2r[...], b2r[...], gr[...], br[...])
        outr[...] = e + y

    row = lambda i: (i, 0)
    fix = lambda i: (0, 0)
    in_specs = [pl.BlockSpec((_RBLK, din), row),
                pl.BlockSpec((_RBLK, _HID // 2), row),
                pl.BlockSpec((_RBLK, _HID // 2), row)]
    args = [ef, gs, gd]
    for w, shp in [(enc_p["W1"], None), (enc_p["b1"].reshape(1, _HID), None),
                   (enc_p["W2"], None), (enc_p["b2"].reshape(1, _HID), None),
                   (enc_p["g"].reshape(1, _HID), None),
                   (enc_p["b"].reshape(1, _HID), None),
                   (edge_p["W1"][:_HID], None),
                   (edge_p["b1"].reshape(1, _HID), None),
                   (edge_p["W2"], None), (edge_p["b2"].reshape(1, _HID), None),
                   (edge_p["g"].reshape(1, _HID), None),
                   (edge_p["b"].reshape(1, _HID), None)]:
        in_specs.append(pl.BlockSpec(w.shape, fix))
        args.append(w)
    return pl.pallas_call(
        body, grid=(np_ // _RBLK,), in_specs=in_specs,
        out_specs=pl.BlockSpec((_RBLK, _HID), row),
        out_shape=jax.ShapeDtypeStruct((np_, _HID), jnp.float32),
        interpret=_INTERPRET)(*args)


def _tc_matmul(x, *ws):
    np_, din = x.shape
    nw = len(ws)

    def body(*refs):
        xv = refs[0][...]
        for k in range(nw):
            refs[1 + nw + k][...] = _pack_val(jnp.dot(
                xv, refs[1 + k][...], preferred_element_type=jnp.float32))

    row = lambda i: (i, 0)
    fix = lambda i: (0, 0)
    outs = pl.pallas_call(
        body, grid=(np_ // _RBLK,),
        in_specs=[pl.BlockSpec((_RBLK, din), row)]
                 + [pl.BlockSpec(w.shape, fix) for w in ws],
        out_specs=[pl.BlockSpec((_RBLK, w.shape[1] // 2), row) for w in ws],
        out_shape=[jax.ShapeDtypeStruct((np_, w.shape[1] // 2), jnp.int32)
                   for w in ws],
        interpret=_INTERPRET)(x, *ws)
    return outs if nw > 1 else outs[0]


_NW = 32


def _ds8(start, size, align=8):
    return pl.ds(pl.multiple_of(start, align), size)


def _chunk(bpw, cap=80, even=False):
    for c in (80, 48, 40, 16, 8):
        if c <= cap and bpw % c == 0 and (not even or (bpw // c) % 2 == 0):
            return c
    raise ValueError(bpw)


def _sc_gather2(table_a, idx_a, table_b, idx_b):
    e = idx_a.shape[0]
    d = table_a.shape[1]
    bpw = e // _NW
    cchunk = _chunk(bpw, even=True)
    nchunks = bpw // cchunk
    mesh = plsc.VectorSubcoreMesh(core_axis_name="c", subcore_axis_name="s")
    dt = table_a.dtype

    @functools.partial(
        pl.kernel, mesh=mesh,
        out_type=[jax.ShapeDtypeStruct((e, d), dt)] * 2,
        scratch_types=[pltpu.VMEM((cchunk,), jnp.int32),
                       pltpu.VMEM((cchunk,), jnp.int32),
                       pltpu.VMEM((cchunk,), jnp.int32),
                       pltpu.VMEM((cchunk,), jnp.int32),
                       pltpu.VMEM((cchunk, d), dt),
                       pltpu.VMEM((cchunk, d), dt),
                       pltpu.VMEM((cchunk, d), dt),
                       pltpu.VMEM((cchunk, d), dt)]
                      + [pltpu.SemaphoreType.DMA] * 8)
    def k(ta, ia, tb, ib, oa, ob, iva0, iva1, ivb0, ivb1,
          ra0, ra1, rb0, rb1,
          g0a, g0b, g1a, g1b, w0a, w0b, w1a, w1b):
        wid = lax.axis_index("s") * 2 + lax.axis_index("c")
        ivas, ivbs = (iva0, iva1), (ivb0, ivb1)
        ras, rbs = (ra0, ra1), (rb0, rb1)
        gsems = ((g0a, g0b), (g1a, g1b))
        wsems = ((w0a, w0b), (w1a, w1b))

        def start_gather(c, bi):
            base = wid * bpw + c * cchunk
            pltpu.sync_copy(ia.at[_ds8(base, cchunk)], ivas[bi])
            pltpu.sync_copy(ib.at[_ds8(base, cchunk)], ivbs[bi])
            pltpu.async_copy(ta.at[ivas[bi]], ras[bi], gsems[bi][0])
            pltpu.async_copy(tb.at[ivbs[bi]], rbs[bi], gsems[bi][1])

        def wait_gather(bi):
            pltpu.make_async_copy(ta.at[ivas[bi]], ras[bi],
                                  gsems[bi][0]).wait()
            pltpu.make_async_copy(tb.at[ivbs[bi]], rbs[bi],
                                  gsems[bi][1]).wait()

        def start_write(c, bi):
            base = wid * bpw + c * cchunk
            pltpu.async_copy(ras[bi], oa.at[_ds8(base, cchunk)], wsems[bi][0])
            pltpu.async_copy(rbs[bi], ob.at[_ds8(base, cchunk)], wsems[bi][1])

        def wait_write(bi):
            pltpu.make_async_copy(ras[bi], oa.at[pl.ds(0, cchunk)],
                                  wsems[bi][0]).wait()
            pltpu.make_async_copy(rbs[bi], ob.at[pl.ds(0, cchunk)],
                                  wsems[bi][1]).wait()

        start_gather(0, 0)

        def body(i2, _):
            c0 = 2 * i2
            c1 = c0 + 1
            c2 = c0 + 2

            @pl.when(i2 > 0)
            def _():
                wait_write(1)

            start_gather(c1, 1)
            wait_gather(0)
            start_write(c0, 0)
            wait_gather(1)
            start_write(c1, 1)

            @pl.when(c2 < nchunks)
            def _():
                wait_write(0)
                start_gather(c2, 0)

            return 0

        lax.fori_loop(0, nchunks // 2, body, 0)
        wait_write(0)
        wait_write(1)

    return k(table_a, idx_a, table_b, idx_b)


def _sc_segsum(rows, dst, n_out):
    if n_out * 128 * 4 <= 7 << 20:
        return _sc_segsum_small(rows, dst, n_out)
    return _sc_segsum_rows(rows, dst, n_out)


def _sc_segsum_small(rows, dst, n_out):
    e = rows.shape[0]
    dt = rows.dtype
    al = 16 if dt == jnp.bfloat16 else 8
    bpw = e // 16
    cchunk = _chunk(bpw, even=True)
    nchunks = bpw // cchunk
    ntile_rows = n_out // 16
    assert n_out % 16 == 0
    zeros = jnp.zeros((n_out, 128), dt)
    mesh = plsc.VectorSubcoreMesh(core_axis_name="c", subcore_axis_name="s")

    @functools.partial(
        pl.kernel, mesh=mesh,
        out_type=jax.ShapeDtypeStruct((2, n_out, 128), dt),
        scratch_types=[pltpu.VMEM((cchunk,), jnp.int32),
                       pltpu.VMEM((cchunk,), jnp.int32),
                       pltpu.VMEM((cchunk, 128), dt),
                       pltpu.VMEM((cchunk, 128), dt),
                       pltpu.VMEM_SHARED((n_out, 128), dt)]
                      + [pltpu.SemaphoreType.DMA] * 4)
    def k(rows_h, dst_h, zeros_h, out_h, iv0, iv1, b0, b1, acc_s,
          si0, si1, sr0, sr1):
        cid = lax.axis_index("c")
        sid = lax.axis_index("s")
        ivs, bufs = (iv0, iv1), (b0, b1)
        isems, rsems = (si0, si1), (sr0, sr1)
        tslice = _ds8(sid * ntile_rows, ntile_rows, al)

        pltpu.sync_copy(zeros_h.at[tslice], acc_s.at[tslice])
        plsc.subcore_barrier()

        def start_dma(c, bi):
            base = sid * bpw + c * cchunk
            pltpu.async_copy(dst_h.at[_ds8(base, cchunk)], ivs[bi], isems[bi])
            pltpu.async_copy(
                rows_h.at[_ds8(base, cchunk, al), _ds8(cid * 128, 128)],
                bufs[bi], rsems[bi])

        def wait_dma(bi):
            pltpu.make_async_copy(dst_h.at[pl.ds(0, cchunk)], ivs[bi],
                                  isems[bi]).wait()
            pltpu.make_async_copy(rows_h.at[pl.ds(0, cchunk),
                                            pl.ds(0, 128)],
                                  bufs[bi], rsems[bi]).wait()

        start_dma(0, 0)

        def body(i2, _):
            c2 = 2 * i2 + 2
            start_dma(2 * i2 + 1, 1)
            wait_dma(0)
            pltpu.sync_copy(b0, acc_s.at[iv0], add=True)

            @pl.when(c2 < nchunks)
            def _():
                start_dma(c2, 0)

            wait_dma(1)
            pltpu.sync_copy(b1, acc_s.at[iv1], add=True)
            return 0

        lax.fori_loop(0, nchunks // 2, body, 0)
        plsc.subcore_barrier()
        pltpu.sync_copy(acc_s.at[tslice], out_h.at[cid, tslice])

    out = k(rows, dst, zeros)
    return out.transpose(1, 0, 2).reshape(n_out, 256)


def _sc_segsum_rows(rows, dst, n_out):
    e = rows.shape[0]
    dt = rows.dtype
    al = 16 if dt == jnp.bfloat16 else 8
    bpw = e // 16
    cchunk = 80
    assert bpw % cchunk == 0 and (bpw // cchunk) % 2 == 0
    nchunks = bpw // cchunk
    rp = 26368 if dt == jnp.bfloat16 else 13184
    npass = -(-n_out // rp)
    acc_rows = rp + 256
    zeros = jnp.zeros((acc_rows, 128), dt)
    mesh = plsc.VectorSubcoreMesh(core_axis_name="c", subcore_axis_name="s")

    @functools.partial(
        pl.kernel, mesh=mesh,
        out_type=jax.ShapeDtypeStruct((2, npass * rp, 128), dt),
        scratch_types=[pltpu.VMEM((cchunk,), jnp.int32),
                       pltpu.VMEM((cchunk,), jnp.int32),
                       pltpu.VMEM((cchunk,), jnp.int32),
                       pltpu.VMEM((cchunk,), jnp.int32),
                       pltpu.VMEM((cchunk, 128), dt),
                       pltpu.VMEM((cchunk, 128), dt),
                       pltpu.VMEM_SHARED((acc_rows, 128), dt)]
                      + [pltpu.SemaphoreType.DMA] * 4)
    def k(rows_h, dst_h, zeros_h, out_h, iv0, iv1, ix0, ix1, b0, b1, acc_s,
          si0, si1, sr0, sr1):
        cid = lax.axis_index("c")
        sid = lax.axis_index("s")
        ivs, ixs, bufs = (iv0, iv1), (ix0, ix1), (b0, b1)
        isems, rsems = (si0, si1), (sr0, sr1)
        zslice = _ds8(sid * (acc_rows // 16), acc_rows // 16, al)

        def start_dma(c, bi):
            base = sid * bpw + c * cchunk
            pltpu.async_copy(dst_h.at[_ds8(base, cchunk)], ivs[bi], isems[bi])
            pltpu.async_copy(
                rows_h.at[_ds8(base, cchunk, al), _ds8(cid * 128, 128)],
                bufs[bi], rsems[bi])

        def wait_dma(bi):
            pltpu.make_async_copy(dst_h.at[pl.ds(0, cchunk)], ivs[bi],
                                  isems[bi]).wait()
            pltpu.make_async_copy(rows_h.at[pl.ds(0, cchunk), pl.ds(0, 128)],
                                  bufs[bi], rsems[bi]).wait()

        def rebase_scatter(bi, seg0):
            for j in range(cchunk // 16):
                v = ivs[bi][pl.ds(j * 16, 16)]
                local = v - seg0
                ok = (local >= 0) & (local < rp)
                ixs[bi][pl.ds(j * 16, 16)] = jnp.where(ok, local, rp)
            pltpu.sync_copy(bufs[bi], acc_s.at[ixs[bi]], add=True)

        for pp in range(npass):
            seg0 = pp * rp
            pltpu.sync_copy(zeros_h.at[zslice], acc_s.at[zslice])
            plsc.subcore_barrier()
            start_dma(0, 0)

            def body(i2, _, seg0=seg0):
                c2 = 2 * i2 + 2
                start_dma(2 * i2 + 1, 1)
                wait_dma(0)
                rebase_scatter(0, seg0)

                @pl.when(c2 < nchunks)
                def _():
                    start_dma(c2, 0)

                wait_dma(1)
                rebase_scatter(1, seg0)
                return 0

            lax.fori_loop(0, nchunks // 2, body, 0)
            plsc.subcore_barrier()
            pltpu.sync_copy(acc_s.at[_ds8(sid * (rp // 16), rp // 16, al)],
                            out_h.at[cid, _ds8(seg0 + sid * (rp // 16),
                                               rp // 16, al)])
            plsc.subcore_barrier()

    out = k(rows, dst, zeros)
    return out.transpose(1, 0, 2).reshape(npass * rp, 256)[:n_out]


_USE_SC_GATHER = True
_USE_SC_SEGSUM = True


def _gather2(ta, ia, tb, ib):
    if _USE_SC_GATHER:
        return _sc_gather2(ta, ia, tb, ib)
    return ta[ia], tb[ib]


def _segsum(rows, dst, n_out):
    if _USE_SC_SEGSUM:
        return _sc_segsum(rows, dst, n_out)
    return jax.ops.segment_sum(rows, dst, num_segments=n_out)


def kernel(x, edge_g2m, edge_mesh, edge_m2g, params):
    p = params
    grid_in = _pad_rows(x.reshape(_C_IN, _N_GRID).T, _NP_GRID)
    mesh_in = _pad_rows(_pad_cols(p["mesh_nfeat"], 8), _NP_MESH)

    src_g, dst_g = edge_g2m[0], edge_g2m[1]
    src_g = jnp.pad(src_g, (0, _NP_EG - src_g.shape[0]))
    dst_g = jnp.pad(dst_g, (0, _NP_EG - dst_g.shape[0]),
                    constant_values=_N_MESH)
    ms, md = edge_mesh[0], edge_mesh[1]
    ms = jnp.pad(ms, (0, _NP_EM - ms.shape[0]))
    md = jnp.pad(md, (0, _NP_EM - md.shape[0]), constant_values=_N_MESH)
    s3, d3 = edge_m2g[0], edge_m2g[1]
    s3 = jnp.pad(s3, (0, _NP_EG - s3.shape[0]))
    d3 = jnp.pad(d3, (0, _NP_EG - d3.shape[0]), constant_values=_N_GRID)

    ef_g2m = _pad_rows(_pad_cols(p["efeat_g2m"], 8), _NP_EG)
    ef_mesh = _pad_rows(_pad_cols(p["efeat_mesh"], 8), _NP_EM)
    ef_m2g = _pad_rows(_pad_cols(p["efeat_m2g"], 8), _NP_EG)

    enc_grid = dict(p["enc_grid"])
    enc_mesh = dict(p["enc_mesh"])
    enc_mesh = {**enc_mesh, "W1": jnp.pad(enc_mesh["W1"], ((0, 5), (0, 0)))}
    enc_eg2m = {**p["enc_eg2m"],
                "W1": jnp.pad(p["enc_eg2m"]["W1"], ((0, 4), (0, 0)))}
    enc_emesh = {**p["enc_emesh"],
                 "W1": jnp.pad(p["enc_emesh"]["W1"], ((0, 4), (0, 0)))}
    enc_em2g = {**p["enc_em2g"],
                "W1": jnp.pad(p["enc_em2g"]["W1"], ((0, 4), (0, 0)))}

    w_g2m = p["g2m_edge"]["W1"]
    gfeat, ts = _tc_mlp([grid_in], [enc_grid["W1"]], [], enc_grid,
                        extra_out_w=w_g2m[_HID:2 * _HID])
    mfeat, td = _tc_mlp([mesh_in], [enc_mesh["W1"]], [], enc_mesh,
                        extra_out_w=w_g2m[2 * _HID:])
    e2 = _tc_mlp([ef_mesh], [enc_emesh["W1"]], [], enc_emesh)

    gs, gd = _gather2(ts, src_g, td, dst_g)
    e1 = _tc_edge_fused(ef_g2m, enc_eg2m, gs, gd, p["g2m_edge"])
    agg = _segsum(e1, dst_g, _NP_MESH)
    wn = p["g2m_node"]["W1"]
    mfeat = _tc_mlp([mfeat, agg], [wn[:_HID], wn[_HID:]], [], p["g2m_node"],
                    res_is_x0=True)
    gfeat = _tc_mlp([gfeat], [p["g2m_grid"]["W1"]], [], p["g2m_grid"],
                    res_is_x0=True)

    for lp in p["proc"]:
        w1 = lp["edge"]["W1"]
        ts, td = _tc_matmul(mfeat, w1[_HID:2 * _HID], w1[2 * _HID:])
        gs, gd = _gather2(ts, ms, td, md)
        e2 = _tc_mlp([e2], [w1[:_HID]], [gs, gd], lp["edge"],
                     res_is_x0=True)
        agg = _segsum(e2, md, _NP_MESH)
        wn = lp["node"]["W1"]
        mfeat = _tc_mlp([mfeat, agg], [wn[:_HID], wn[_HID:]], [], lp["node"],
                        res_is_x0=True)

    w1 = p["m2g_edge"]["W1"]
    ts = _tc_matmul(mfeat, w1[_HID:2 * _HID])
    td = _tc_matmul(gfeat, w1[2 * _HID:])
    gs, gd = _gather2(ts, s3, td, d3)
    e3 = _tc_edge_fused(ef_m2g, enc_em2g, gs, gd, p["m2g_edge"])
    agg = _segsum(e3, d3, _NP_GRID)
    wn = p["m2g_node"]["W1"]
    gfeat = _tc_mlp([gfeat, agg], [wn[:_HID], wn[_HID:]], [], p["m2g_node"],
                    res_is_x0=True)
    out = _tc_mlp([gfeat], [p["dec_out"]["W1"]], [], p["dec_out"], ln=False)
    return out[:_N_GRID].T.reshape(1, _C_OUT, _H, _W)


# speedup vs baseline: 2.6892x; 1.0097x over previous
import functools

import jax
import jax.numpy as jnp
from jax import lax
from jax.experimental import pallas as pl
from jax.experimental.pallas import tpu as pltpu
from jax.experimental.pallas import tpu_sc as plsc

_H, _W = 181, 360
_N_GRID = _H * _W
_N_MESH = 10242
_C_IN = 128
_C_OUT = 128
_HID = 256
_L = 4

_NP_GRID = 65536
_NP_MESH = 10752
_NP_EG = 130560
_NP_EM = 40960

_RBLK = 512


def _pad_rows(a, n):
    return jnp.pad(a, ((0, n - a.shape[0]), (0, 0)))


def _pad_cols(a, n):
    return jnp.pad(a, ((0, 0), (0, n - a.shape[1])))


def _pack_val(t):
    d = t.shape[1] // 2
    a = lax.bitcast_convert_type(t[:, :d], jnp.int32)
    b = lax.bitcast_convert_type(t[:, d:], jnp.int32)
    a = ((a + 0x8000) >> 16) & 0xFFFF
    b = (b + 0x8000) & jnp.int32(-65536)
    return a | b


def _unpack_val(g):
    lo = lax.bitcast_convert_type(g << 16, jnp.float32)
    hi = lax.bitcast_convert_type(g & jnp.int32(-65536), jnp.float32)
    return jnp.concatenate([lo, hi], axis=1)


def _mlp_val(terms, adds, b1, w2, b2, g, b):
    terms = [(x.astype(jnp.float32), w) for x, w in terms]
    acc = jnp.dot(terms[0][0], terms[0][1], preferred_element_type=jnp.float32)
    for xv, wv in terms[1:]:
        acc = acc + jnp.dot(xv, wv, preferred_element_type=jnp.float32)
    for av in adds:
        acc = acc + av
    acc = acc + b1
    hv = acc * lax.logistic(acc)
    yv = jnp.dot(hv, w2, preferred_element_type=jnp.float32) + b2
    if g is not None:
        mu = jnp.mean(yv, axis=-1, keepdims=True)
        var = jnp.mean((yv - mu) ** 2, axis=-1, keepdims=True)
        yv = (yv - mu) * lax.rsqrt(var + 1e-5) * g + b
    return yv


def _tc_mlp(xs, w1s, adds, p, ln=True, res=None, res_is_x0=False,
            extra_out_w=None):
    np_ = (xs + adds)[0].shape[0]
    dh = w1s[0].shape[1] if w1s else adds[0].shape[1]
    dout = p["W2"].shape[1]
    nx, na = len(xs), len(adds)
    has_res = res is not None or res_is_x0
    n_extra = 1 if extra_out_w is not None else 0

    def body(*refs):
        i = 0
        xr = refs[:nx]; i += nx
        ar = refs[i:i + na]; i += na
        wr = refs[i:i + nx]; i += nx
        b1r = refs[i]; w2r = refs[i + 1]; b2r = refs[i + 2]; i += 3
        gr = br = None
        if ln:
            gr, br = refs[i], refs[i + 1]; i += 2
        rr = None
        if res is not None:
            rr = refs[i]; i += 1
        ewr = None
        if extra_out_w is not None:
            ewr = refs[i]; i += 1
        outr = refs[i]
        out2r = refs[i + 1] if n_extra else None
        y = _mlp_val([(xr[k][...], wr[k][...]) for k in range(nx)],
                     [_unpack_val(a[...]) for a in ar], b1r[...],
                     w2r[...], b2r[...], gr[...] if ln else None,
                     br[...] if ln else None)
        if res_is_x0:
            y = y + xr[0][...]
        elif rr is not None:
            y = y + rr[...]
        outr[...] = y
        if extra_out_w is not None:
            out2r[...] = _pack_val(jnp.dot(
                y, ewr[...], preferred_element_type=jnp.float32))

    grid = (np_ // _RBLK,)
    row = lambda i: (i, 0)
    fix = lambda i: (0, 0)
    in_specs = [pl.BlockSpec((_RBLK, x.shape[1]), row) for x in xs]
    in_specs += [pl.BlockSpec((_RBLK, dh // 2), row) for _ in adds]
    in_specs += [pl.BlockSpec(w.shape, fix) for w in w1s]
    args = list(xs) + list(adds) + list(w1s)
    b1 = p["b1"].reshape(1, dh)
    w2 = p["W2"]
    b2 = p["b2"].reshape(1, dout)
    in_specs += [pl.BlockSpec((1, dh), fix), pl.BlockSpec(w2.shape, fix),
                 pl.BlockSpec((1, dout), fix)]
    args += [b1, w2, b2]
    if ln:
        in_specs += [pl.BlockSpec((1, dout), fix), pl.BlockSpec((1, dout), fix)]
        args += [p["g"].reshape(1, dout), p["b"].reshape(1, dout)]
    if res is not None:
        in_specs += [pl.BlockSpec((_RBLK, dout), row)]
        args += [res]
    out_shape = [jax.ShapeDtypeStruct((np_, dout), jnp.float32)]
    out_specs = [pl.BlockSpec((_RBLK, dout), row)]
    if extra_out_w is not None:
        in_specs += [pl.BlockSpec(extra_out_w.shape, fix)]
        args += [extra_out_w]
        dk = extra_out_w.shape[1] // 2
        out_shape += [jax.ShapeDtypeStruct((np_, dk), jnp.int32)]
        out_specs += [pl.BlockSpec((_RBLK, dk), row)]
    outs = pl.pallas_call(
        body, grid=grid, in_specs=in_specs, out_specs=out_specs,
        out_shape=out_shape)(*args)
    return outs if n_extra else outs[0]


def _tc_edge_fused(ef, enc_p, gs, gd, edge_p):
    np_ = ef.shape[0]
    din = ef.shape[1]

    def body(efr, gsr, gdr,
             ew1, eb1, ew2, eb2, eg, ebb,
             w1a, b1r, w2r, b2r, gr, br, outr):
        e = _mlp_val([(efr[...], ew1[...])], [], eb1[...], ew2[...], eb2[...],
                     eg[...], ebb[...])
        y = _mlp_val([(e, w1a[...])],
                     [_unpack_val(gsr[...]), _unpack_val(gdr[...])],
                     b1r[...], w2r[...], b2r[...], gr[...], br[...])
        outr[...] = e + y

    row = lambda i: (i, 0)
    fix = lambda i: (0, 0)
    in_specs = [pl.BlockSpec((_RBLK, din), row),
                pl.BlockSpec((_RBLK, _HID // 2), row),
                pl.BlockSpec((_RBLK, _HID // 2), row)]
    args = [ef, gs, gd]
    for w, shp in [(enc_p["W1"], None), (enc_p["b1"].reshape(1, _HID), None),
                   (enc_p["W2"], None), (enc_p["b2"].reshape(1, _HID), None),
                   (enc_p["g"].reshape(1, _HID), None),
                   (enc_p["b"].reshape(1, _HID), None),
                   (edge_p["W1"][:_HID], None),
                   (edge_p["b1"].reshape(1, _HID), None),
                   (edge_p["W2"], None), (edge_p["b2"].reshape(1, _HID), None),
                   (edge_p["g"].reshape(1, _HID), None),
                   (edge_p["b"].reshape(1, _HID), None)]:
        in_specs.append(pl.BlockSpec(w.shape, fix))
        args.append(w)
    return pl.pallas_call(
        body, grid=(np_ // _RBLK,), in_specs=in_specs,
        out_specs=pl.BlockSpec((_RBLK, _HID), row),
        out_shape=jax.ShapeDtypeStruct((np_, _HID), jnp.float32),
        )(*args)


def _tc_matmul(x, *ws):
    np_, din = x.shape
    nw = len(ws)

    def body(*refs):
        xv = refs[0][...]
        for k in range(nw):
            refs[1 + nw + k][...] = _pack_val(jnp.dot(
                xv, refs[1 + k][...], preferred_element_type=jnp.float32))

    row = lambda i: (i, 0)
    fix = lambda i: (0, 0)
    outs = pl.pallas_call(
        body, grid=(np_ // _RBLK,),
        in_specs=[pl.BlockSpec((_RBLK, din), row)]
                 + [pl.BlockSpec(w.shape, fix) for w in ws],
        out_specs=[pl.BlockSpec((_RBLK, w.shape[1] // 2), row) for w in ws],
        out_shape=[jax.ShapeDtypeStruct((np_, w.shape[1] // 2), jnp.int32)
                   for w in ws],
        )(x, *ws)
    return outs if nw > 1 else outs[0]


_NW = 32


def _ds8(start, size, align=8):
    return pl.ds(pl.multiple_of(start, align), size)


def _chunk(bpw, cap=80, even=False):
    for c in (80, 48, 40, 16, 8):
        if c <= cap and bpw % c == 0 and (not even or (bpw // c) % 2 == 0):
            return c
    raise ValueError(bpw)


def _sc_gather2(table_a, idx_a, table_b, idx_b):
    e = idx_a.shape[0]
    d = table_a.shape[1]
    bpw = e // _NW
    cchunk = _chunk(bpw, even=True)
    nchunks = bpw // cchunk
    mesh = plsc.VectorSubcoreMesh(core_axis_name="c", subcore_axis_name="s")
    dt = table_a.dtype

    @functools.partial(
        pl.kernel, mesh=mesh,
        out_type=[jax.ShapeDtypeStruct((e, d), dt)] * 2,
        scratch_types=[pltpu.VMEM((cchunk,), jnp.int32),
                       pltpu.VMEM((cchunk,), jnp.int32),
                       pltpu.VMEM((cchunk,), jnp.int32),
                       pltpu.VMEM((cchunk,), jnp.int32),
                       pltpu.VMEM((cchunk, d), dt),
                       pltpu.VMEM((cchunk, d), dt),
                       pltpu.VMEM((cchunk, d), dt),
                       pltpu.VMEM((cchunk, d), dt)]
                      + [pltpu.SemaphoreType.DMA] * 8)
    def k(ta, ia, tb, ib, oa, ob, iva0, iva1, ivb0, ivb1,
          ra0, ra1, rb0, rb1,
          g0a, g0b, g1a, g1b, w0a, w0b, w1a, w1b):
        wid = lax.axis_index("s") * 2 + lax.axis_index("c")
        ivas, ivbs = (iva0, iva1), (ivb0, ivb1)
        ras, rbs = (ra0, ra1), (rb0, rb1)
        gsems = ((g0a, g0b), (g1a, g1b))
        wsems = ((w0a, w0b), (w1a, w1b))

        def start_gather(c, bi):
            base = wid * bpw + c * cchunk
            pltpu.sync_copy(ia.at[_ds8(base, cchunk)], ivas[bi])
            pltpu.sync_copy(ib.at[_ds8(base, cchunk)], ivbs[bi])
            pltpu.async_copy(ta.at[ivas[bi]], ras[bi], gsems[bi][0])
            pltpu.async_copy(tb.at[ivbs[bi]], rbs[bi], gsems[bi][1])

        def wait_gather(bi):
            pltpu.make_async_copy(ta.at[ivas[bi]], ras[bi],
                                  gsems[bi][0]).wait()
            pltpu.make_async_copy(tb.at[ivbs[bi]], rbs[bi],
                                  gsems[bi][1]).wait()

        def start_write(c, bi):
            base = wid * bpw + c * cchunk
            pltpu.async_copy(ras[bi], oa.at[_ds8(base, cchunk)], wsems[bi][0])
            pltpu.async_copy(rbs[bi], ob.at[_ds8(base, cchunk)], wsems[bi][1])

        def wait_write(bi):
            pltpu.make_async_copy(ras[bi], oa.at[pl.ds(0, cchunk)],
                                  wsems[bi][0]).wait()
            pltpu.make_async_copy(rbs[bi], ob.at[pl.ds(0, cchunk)],
                                  wsems[bi][1]).wait()

        start_gather(0, 0)

        def body(i2, _):
            c0 = 2 * i2
            c1 = c0 + 1
            c2 = c0 + 2

            @pl.when(i2 > 0)
            def _():
                wait_write(1)

            start_gather(c1, 1)
            wait_gather(0)
            start_write(c0, 0)
            wait_gather(1)
            start_write(c1, 1)

            @pl.when(c2 < nchunks)
            def _():
                wait_write(0)
                start_gather(c2, 0)

            return 0

        lax.fori_loop(0, nchunks // 2, body, 0)
        wait_write(0)
        wait_write(1)

    return k(table_a, idx_a, table_b, idx_b)


def _sc_segsum(rows, dst, n_out):
    if n_out * 128 * 4 <= 7 << 20:
        return _sc_segsum_small(rows, dst, n_out)
    return _sc_segsum_rows(rows, dst, n_out)


def _sc_segsum_small(rows, dst, n_out):
    e = rows.shape[0]
    dt = rows.dtype
    al = 16 if dt == jnp.bfloat16 else 8
    bpw = e // 16
    cchunk = _chunk(bpw, even=True)
    nchunks = bpw // cchunk
    ntile_rows = n_out // 16
    assert n_out % 16 == 0
    zeros = jnp.zeros((n_out, 128), dt)
    mesh = plsc.VectorSubcoreMesh(core_axis_name="c", subcore_axis_name="s")

    @functools.partial(
        pl.kernel, mesh=mesh,
        out_type=jax.ShapeDtypeStruct((2, n_out, 128), dt),
        scratch_types=[pltpu.VMEM((cchunk,), jnp.int32),
                       pltpu.VMEM((cchunk,), jnp.int32),
                       pltpu.VMEM((cchunk, 128), dt),
                       pltpu.VMEM((cchunk, 128), dt),
                       pltpu.VMEM_SHARED((n_out, 128), dt)]
                      + [pltpu.SemaphoreType.DMA] * 4)
    def k(rows_h, dst_h, zeros_h, out_h, iv0, iv1, b0, b1, acc_s,
          si0, si1, sr0, sr1):
        cid = lax.axis_index("c")
        sid = lax.axis_index("s")
        ivs, bufs = (iv0, iv1), (b0, b1)
        isems, rsems = (si0, si1), (sr0, sr1)
        tslice = _ds8(sid * ntile_rows, ntile_rows, al)

        pltpu.sync_copy(zeros_h.at[tslice], acc_s.at[tslice])
        plsc.subcore_barrier()

        def start_dma(c, bi):
            base = sid * bpw + c * cchunk
            pltpu.async_copy(dst_h.at[_ds8(base, cchunk)], ivs[bi], isems[bi])
            pltpu.async_copy(
                rows_h.at[_ds8(base, cchunk, al), _ds8(cid * 128, 128)],
                bufs[bi], rsems[bi])

        def wait_dma(bi):
            pltpu.make_async_copy(dst_h.at[pl.ds(0, cchunk)], ivs[bi],
                                  isems[bi]).wait()
            pltpu.make_async_copy(rows_h.at[pl.ds(0, cchunk),
                                            pl.ds(0, 128)],
                                  bufs[bi], rsems[bi]).wait()

        start_dma(0, 0)

        def body(i2, _):
            c2 = 2 * i2 + 2
            start_dma(2 * i2 + 1, 1)
            wait_dma(0)
            pltpu.sync_copy(b0, acc_s.at[iv0], add=True)

            @pl.when(c2 < nchunks)
            def _():
                start_dma(c2, 0)

            wait_dma(1)
            pltpu.sync_copy(b1, acc_s.at[iv1], add=True)
            return 0

        lax.fori_loop(0, nchunks // 2, body, 0)
        plsc.subcore_barrier()
        pltpu.sync_copy(acc_s.at[tslice], out_h.at[cid, tslice])

    out = k(rows, dst, zeros)
    return out.transpose(1, 0, 2).reshape(n_out, 256)


def _sc_segsum_rows(rows, dst, n_out):
    e = rows.shape[0]
    dt = rows.dtype
    al = 16 if dt == jnp.bfloat16 else 8
    bpw = e // 16
    cchunk = 80
    assert bpw % cchunk == 0 and (bpw // cchunk) % 2 == 0
    nchunks = bpw // cchunk
    rp = 26368 if dt == jnp.bfloat16 else 13184
    npass = -(-n_out // rp)
    acc_rows = rp + 256
    zeros = jnp.zeros((acc_rows, 128), dt)
    mesh = plsc.VectorSubcoreMesh(core_axis_name="c", subcore_axis_name="s")

    @functools.partial(
        pl.kernel, mesh=mesh,
        out_type=jax.ShapeDtypeStruct((2, npass * rp, 128), dt),
        scratch_types=[pltpu.VMEM((cchunk,), jnp.int32),
                       pltpu.VMEM((cchunk,), jnp.int32),
                       pltpu.VMEM((cchunk,), jnp.int32),
                       pltpu.VMEM((cchunk,), jnp.int32),
                       pltpu.VMEM((cchunk, 128), dt),
                       pltpu.VMEM((cchunk, 128), dt),
                       pltpu.VMEM_SHARED((acc_rows, 128), dt)]
                      + [pltpu.SemaphoreType.DMA] * 4)
    def k(rows_h, dst_h, zeros_h, out_h, iv0, iv1, ix0, ix1, b0, b1, acc_s,
          si0, si1, sr0, sr1):
        cid = lax.axis_index("c")
        sid = lax.axis_index("s")
        ivs, ixs, bufs = (iv0, iv1), (ix0, ix1), (b0, b1)
        isems, rsems = (si0, si1), (sr0, sr1)
        zslice = _ds8(sid * (acc_rows // 16), acc_rows // 16, al)

        def start_dma(c, bi):
            base = sid * bpw + c * cchunk
            pltpu.async_copy(dst_h.at[_ds8(base, cchunk)], ivs[bi], isems[bi])
            pltpu.async_copy(
                rows_h.at[_ds8(base, cchunk, al), _ds8(cid * 128, 128)],
                bufs[bi], rsems[bi])

        def wait_dma(bi):
            pltpu.make_async_copy(dst_h.at[pl.ds(0, cchunk)], ivs[bi],
                                  isems[bi]).wait()
            pltpu.make_async_copy(rows_h.at[pl.ds(0, cchunk), pl.ds(0, 128)],
                                  bufs[bi], rsems[bi]).wait()

        def rebase_scatter(bi, seg0):
            for j in range(cchunk // 16):
                v = ivs[bi][pl.ds(j * 16, 16)]
                local = v - seg0
                ok = (local >= 0) & (local < rp)
                ixs[bi][pl.ds(j * 16, 16)] = jnp.where(ok, local, rp)
            pltpu.sync_copy(bufs[bi], acc_s.at[ixs[bi]], add=True)

        for pp in range(npass):
            seg0 = pp * rp
            pltpu.sync_copy(zeros_h.at[zslice], acc_s.at[zslice])
            plsc.subcore_barrier()
            start_dma(0, 0)

            def body(i2, _, seg0=seg0):
                c2 = 2 * i2 + 2
                start_dma(2 * i2 + 1, 1)
                wait_dma(0)
                rebase_scatter(0, seg0)

                @pl.when(c2 < nchunks)
                def _():
                    start_dma(c2, 0)

                wait_dma(1)
                rebase_scatter(1, seg0)
                return 0

            lax.fori_loop(0, nchunks // 2, body, 0)
            plsc.subcore_barrier()
            pltpu.sync_copy(acc_s.at[_ds8(sid * (rp // 16), rp // 16, al)],
                            out_h.at[cid, _ds8(seg0 + sid * (rp // 16),
                                               rp // 16, al)])
            plsc.subcore_barrier()

    out = k(rows, dst, zeros)
    return out.transpose(1, 0, 2).reshape(npass * rp, 256)[:n_out]


_gather2 = _sc_gather2
_segsum = _sc_segsum


def kernel(x, edge_g2m, edge_mesh, edge_m2g, params):
    p = params
    grid_in = _pad_rows(x.reshape(_C_IN, _N_GRID).T, _NP_GRID)
    mesh_in = _pad_rows(_pad_cols(p["mesh_nfeat"], 8), _NP_MESH)

    src_g, dst_g = edge_g2m[0], edge_g2m[1]
    src_g = jnp.pad(src_g, (0, _NP_EG - src_g.shape[0]))
    dst_g = jnp.pad(dst_g, (0, _NP_EG - dst_g.shape[0]),
                    constant_values=_N_MESH)
    ms, md = edge_mesh[0], edge_mesh[1]
    ms = jnp.pad(ms, (0, _NP_EM - ms.shape[0]))
    md = jnp.pad(md, (0, _NP_EM - md.shape[0]), constant_values=_N_MESH)
    s3, d3 = edge_m2g[0], edge_m2g[1]
    s3 = jnp.pad(s3, (0, _NP_EG - s3.shape[0]))
    d3 = jnp.pad(d3, (0, _NP_EG - d3.shape[0]), constant_values=_N_GRID)

    ef_g2m = _pad_rows(_pad_cols(p["efeat_g2m"], 8), _NP_EG)
    ef_mesh = _pad_rows(_pad_cols(p["efeat_mesh"], 8), _NP_EM)
    ef_m2g = _pad_rows(_pad_cols(p["efeat_m2g"], 8), _NP_EG)

    enc_grid = dict(p["enc_grid"])
    enc_mesh = dict(p["enc_mesh"])
    enc_mesh = {**enc_mesh, "W1": jnp.pad(enc_mesh["W1"], ((0, 5), (0, 0)))}
    enc_eg2m = {**p["enc_eg2m"],
                "W1": jnp.pad(p["enc_eg2m"]["W1"], ((0, 4), (0, 0)))}
    enc_emesh = {**p["enc_emesh"],
                 "W1": jnp.pad(p["enc_emesh"]["W1"], ((0, 4), (0, 0)))}
    enc_em2g = {**p["enc_em2g"],
                "W1": jnp.pad(p["enc_em2g"]["W1"], ((0, 4), (0, 0)))}

    w_g2m = p["g2m_edge"]["W1"]
    gfeat, ts = _tc_mlp([grid_in], [enc_grid["W1"]], [], enc_grid,
                        extra_out_w=w_g2m[_HID:2 * _HID])
    mfeat, td = _tc_mlp([mesh_in], [enc_mesh["W1"]], [], enc_mesh,
                        extra_out_w=w_g2m[2 * _HID:])
    e2 = _tc_mlp([ef_mesh], [enc_emesh["W1"]], [], enc_emesh)

    gs, gd = _gather2(ts, src_g, td, dst_g)
    e1 = _tc_edge_fused(ef_g2m, enc_eg2m, gs, gd, p["g2m_edge"])
    agg = _segsum(e1, dst_g, _NP_MESH)
    wn = p["g2m_node"]["W1"]
    mfeat = _tc_mlp([mfeat, agg], [wn[:_HID], wn[_HID:]], [], p["g2m_node"],
                    res_is_x0=True)
    gfeat = _tc_mlp([gfeat], [p["g2m_grid"]["W1"]], [], p["g2m_grid"],
                    res_is_x0=True)

    for lp in p["proc"]:
        w1 = lp["edge"]["W1"]
        ts, td = _tc_matmul(mfeat, w1[_HID:2 * _HID], w1[2 * _HID:])
        gs, gd = _gather2(ts, ms, td, md)
        e2 = _tc_mlp([e2], [w1[:_HID]], [gs, gd], lp["edge"],
                     res_is_x0=True)
        agg = _segsum(e2, md, _NP_MESH)
        wn = lp["node"]["W1"]
        mfeat = _tc_mlp([mfeat, agg], [wn[:_HID], wn[_HID:]], [], lp["node"],
                        res_is_x0=True)

    w1 = p["m2g_edge"]["W1"]
    ts = _tc_matmul(mfeat, w1[_HID:2 * _HID])
    td = _tc_matmul(gfeat, w1[2 * _HID:])
    gs, gd = _gather2(ts, s3, td, d3)
    e3 = _tc_edge_fused(ef_m2g, enc_em2g, gs, gd, p["m2g_edge"])
    agg = _segsum(e3, d3, _NP_GRID)
    wn = p["m2g_node"]["W1"]
    gfeat = _tc_mlp([gfeat, agg], [wn[:_HID], wn[_HID:]], [], p["m2g_node"],
                    res_is_x0=True)
    out = _tc_mlp([gfeat], [p["dec_out"]["W1"]], [], p["dec_out"], ln=False)
    return out[:_N_GRID].T.reshape(1, _C_OUT, _H, _W)
